# Initial kernel scaffold; baseline (speedup 1.0000x reference)
import jax
import jax.numpy as jnp
from jax.experimental import pallas as pl


def kernel(x, z, atomref_weight):
    raise NotImplementedError("write your pallas kernel here")


# speedup vs baseline: 97.8060x; 97.8060x over previous
import jax
import jax.numpy as jnp
from jax import lax
from jax.experimental import pallas as pl
from jax.experimental.pallas import tpu as pltpu
from jax.experimental.pallas import tpu_sc as plsc

N = 4_000_000
L = 16
NW = 32
VEC_LO = 7812
CHUNK_V = 651
CHUNK = CHUNK_V * L
NCHUNKS = VEC_LO // CHUNK_V
TAB_PAD = 128


def _body(x_hbm, z_hbm, tab_hbm, out_hbm, tab_v, z_v, x_v, y_v):
    cid = lax.axis_index("c")
    sid = lax.axis_index("s")
    wid = sid * 2 + cid
    base = (VEC_LO * wid + jnp.minimum(wid, 16)) * L

    pltpu.sync_copy(tab_hbm, tab_v)

    def chunk_body(c, carry):
        off = base + c * CHUNK
        pltpu.sync_copy(z_hbm.at[pl.ds(off, CHUNK)], z_v)
        pltpu.sync_copy(x_hbm.at[pl.ds(off, CHUNK)], x_v)

        def vec_body(j, carry2):
            s = j * L
            zv = z_v[pl.ds(s, L)]
            gv = plsc.load_gather(tab_v, [zv])
            y_v[pl.ds(s, L)] = x_v[pl.ds(s, L)] + gv
            return carry2

        lax.fori_loop(0, CHUNK_V, vec_body, 0)
        pltpu.sync_copy(y_v, out_hbm.at[pl.ds(off, CHUNK)])
        return carry

    lax.fori_loop(0, NCHUNKS, chunk_body, 0)

    @pl.when(wid < 16)
    def _tail():
        off = base + VEC_LO * L
        pltpu.sync_copy(z_hbm.at[pl.ds(off, L)], z_v.at[pl.ds(0, L)])
        pltpu.sync_copy(x_hbm.at[pl.ds(off, L)], x_v.at[pl.ds(0, L)])
        zv = z_v[pl.ds(0, L)]
        gv = plsc.load_gather(tab_v, [zv])
        y_v[pl.ds(0, L)] = x_v[pl.ds(0, L)] + gv
        pltpu.sync_copy(y_v.at[pl.ds(0, L)], out_hbm.at[pl.ds(off, L)])


def kernel(x, z, atomref_weight):
    xf = x.reshape(N)
    zi = z.astype(jnp.int32)
    tab = jnp.pad(atomref_weight.reshape(-1), (0, TAB_PAD - atomref_weight.shape[0]))
    mesh = plsc.VectorSubcoreMesh(core_axis_name="c", subcore_axis_name="s")
    k = pl.kernel(
        _body,
        out_type=jax.ShapeDtypeStruct((N,), jnp.float32),
        mesh=mesh,
        compiler_params=pltpu.CompilerParams(needs_layout_passes=False),
        scratch_types=[
            pltpu.VMEM((TAB_PAD,), jnp.float32),
            pltpu.VMEM((CHUNK,), jnp.int32),
            pltpu.VMEM((CHUNK,), jnp.float32),
            pltpu.VMEM((CHUNK,), jnp.float32),
        ],
    )
    out = k(xf, zi, tab)
    return out.reshape(N, 1)


# speedup vs baseline: 118.3781x; 1.2103x over previous
import jax
import jax.numpy as jnp
from jax import lax
from jax.experimental import pallas as pl
from jax.experimental.pallas import tpu as pltpu
from jax.experimental.pallas import tpu_sc as plsc

N = 4_000_000
L = 16
NW = 32
VEC_LO = 7812
CHUNK_V = 868
CHUNK = CHUNK_V * L
NCHUNKS = VEC_LO // CHUNK_V
UNROLL = 14
TAB_PAD = 128


def _body(x_hbm, z_hbm, tab_hbm, out_hbm, tab_v,
          z0, z1, x0, x1, y0, y1, in0, in1, out0, out1):
    cid = lax.axis_index("c")
    sid = lax.axis_index("s")
    wid = sid * 2 + cid
    base = (VEC_LO * wid + jnp.minimum(wid, 16)) * L

    z_bufs, x_bufs, y_bufs = (z0, z1), (x0, x1), (y0, y1)
    in_sems, out_sems = (in0, in1), (out0, out1)

    pltpu.sync_copy(tab_hbm, tab_v)

    def in_copies(c, b):
        off = base + c * CHUNK
        return (pltpu.make_async_copy(z_hbm.at[pl.ds(off, CHUNK)],
                                      z_bufs[b], in_sems[b]),
                pltpu.make_async_copy(x_hbm.at[pl.ds(off, CHUNK)],
                                      x_bufs[b], in_sems[b]))

    def out_copy(c, b):
        off = base + c * CHUNK
        return pltpu.make_async_copy(y_bufs[b],
                                     out_hbm.at[pl.ds(off, CHUNK)],
                                     out_sems[b])

    for h in in_copies(0, 0):
        h.start()

    for c in range(NCHUNKS):
        b = c % 2
        if c + 1 < NCHUNKS:
            for h in in_copies(c + 1, 1 - b):
                h.start()
        if c >= 2:
            out_copy(c - 2, b).wait()
        for h in in_copies(c, b):
            h.wait()

        zb, xb, yb = z_bufs[b], x_bufs[b], y_bufs[b]

        @plsc.parallel_loop(0, CHUNK, step=L, unroll=UNROLL)
        def _inner(s):
            zv = zb[pl.ds(s, L)]
            gv = plsc.load_gather(tab_v, [zv])
            yb[pl.ds(s, L)] = xb[pl.ds(s, L)] + gv

        out_copy(c, b).start()

    out_copy(NCHUNKS - 2, NCHUNKS % 2).wait()
    out_copy(NCHUNKS - 1, (NCHUNKS - 1) % 2).wait()

    @pl.when(wid < 16)
    def _tail():
        off = base + VEC_LO * L
        pltpu.sync_copy(z_hbm.at[pl.ds(off, L)], z0.at[pl.ds(0, L)])
        pltpu.sync_copy(x_hbm.at[pl.ds(off, L)], x0.at[pl.ds(0, L)])
        zv = z0[pl.ds(0, L)]
        gv = plsc.load_gather(tab_v, [zv])
        y0[pl.ds(0, L)] = x0[pl.ds(0, L)] + gv
        pltpu.sync_copy(y0.at[pl.ds(0, L)], out_hbm.at[pl.ds(off, L)])


def kernel(x, z, atomref_weight):
    xf = x.reshape(N)
    zi = z.astype(jnp.int32)
    tab = jnp.pad(atomref_weight.reshape(-1), (0, TAB_PAD - atomref_weight.shape[0]))
    mesh = plsc.VectorSubcoreMesh(core_axis_name="c", subcore_axis_name="s")
    k = pl.kernel(
        _body,
        out_type=jax.ShapeDtypeStruct((N,), jnp.float32),
        mesh=mesh,
        compiler_params=pltpu.CompilerParams(needs_layout_passes=False),
        scratch_types=[
            pltpu.VMEM((TAB_PAD,), jnp.float32),
            pltpu.VMEM((CHUNK,), jnp.int32),
            pltpu.VMEM((CHUNK,), jnp.int32),
            pltpu.VMEM((CHUNK,), jnp.float32),
            pltpu.VMEM((CHUNK,), jnp.float32),
            pltpu.VMEM((CHUNK,), jnp.float32),
            pltpu.VMEM((CHUNK,), jnp.float32),
            pltpu.SemaphoreType.DMA,
            pltpu.SemaphoreType.DMA,
            pltpu.SemaphoreType.DMA,
            pltpu.SemaphoreType.DMA,
        ],
    )
    out = k(xf, zi, tab)
    return out.reshape(N, 1)


# speedup vs baseline: 310.6188x; 2.6240x over previous
import jax
import jax.numpy as jnp
from jax import lax
from jax.experimental import pallas as pl
from jax.experimental.pallas import tpu as pltpu
from jax.experimental.pallas import tpu_sc as plsc

N = 4_000_000
L = 16
VEC_LO = 7812
CHUNK_V = 868
CHUNK = CHUNK_V * L
NCHUNKS = VEC_LO // CHUNK_V
UNROLL = 14


def _body(z_hbm, tab_hbm, out_hbm, tab_v,
          z0, z1, g0, g1, in0, in1, out0, out1):
    cid = lax.axis_index("c")
    sid = lax.axis_index("s")
    wid = sid * 2 + cid
    base = (VEC_LO * wid + jnp.minimum(wid, 16)) * L

    z_bufs, g_bufs = (z0, z1), (g0, g1)
    in_sems, out_sems = (in0, in1), (out0, out1)

    pltpu.sync_copy(tab_hbm, tab_v)

    def in_copy(c, b):
        off = base + c * CHUNK
        return pltpu.make_async_copy(z_hbm.at[pl.ds(off, CHUNK)],
                                     z_bufs[b], in_sems[b])

    def out_copy(c, b):
        off = base + c * CHUNK
        return pltpu.make_async_copy(g_bufs[b],
                                     out_hbm.at[pl.ds(off, CHUNK)],
                                     out_sems[b])

    in_copy(0, 0).start()

    for c in range(NCHUNKS):
        b = c % 2
        if c + 1 < NCHUNKS:
            in_copy(c + 1, 1 - b).start()
        if c >= 2:
            out_copy(c - 2, b).wait()
        in_copy(c, b).wait()

        zb, gb = z_bufs[b], g_bufs[b]

        @plsc.parallel_loop(0, CHUNK, step=L, unroll=UNROLL)
        def _inner(s):
            zv = zb[pl.ds(s, L)]
            gb[pl.ds(s, L)] = plsc.load_gather(tab_v, [zv])

        out_copy(c, b).start()

    out_copy(NCHUNKS - 2, NCHUNKS % 2).wait()
    out_copy(NCHUNKS - 1, (NCHUNKS - 1) % 2).wait()

    @pl.when(wid < 16)
    def _tail():
        off = base + VEC_LO * L
        pltpu.sync_copy(z_hbm.at[pl.ds(off, L)], z0.at[pl.ds(0, L)])
        zv = z0[pl.ds(0, L)]
        g0[pl.ds(0, L)] = plsc.load_gather(tab_v, [zv])
        pltpu.sync_copy(g0.at[pl.ds(0, L)], out_hbm.at[pl.ds(off, L)])


def kernel(x, z, atomref_weight):
    zi = z.astype(jnp.int32)
    tab = jnp.pad(atomref_weight.reshape(-1), (0, 28))
    mesh = plsc.VectorSubcoreMesh(core_axis_name="c", subcore_axis_name="s")
    k = pl.kernel(
        _body,
        out_type=jax.ShapeDtypeStruct((N,), jnp.float32),
        mesh=mesh,
        compiler_params=pltpu.CompilerParams(
            needs_layout_passes=False, use_tc_tiling_on_sc=False),
        scratch_types=[
            pltpu.VMEM((128,), jnp.float32),
            pltpu.VMEM((CHUNK,), jnp.int32),
            pltpu.VMEM((CHUNK,), jnp.int32),
            pltpu.VMEM((CHUNK,), jnp.float32),
            pltpu.VMEM((CHUNK,), jnp.float32),
            pltpu.SemaphoreType.DMA,
            pltpu.SemaphoreType.DMA,
            pltpu.SemaphoreType.DMA,
            pltpu.SemaphoreType.DMA,
        ],
    )
    g = k(zi, tab)
    return x + g.reshape(N, 1)


# speedup vs baseline: 312.5005x; 1.0061x over previous
import jax
import jax.numpy as jnp
from jax import lax
from jax.experimental import pallas as pl
from jax.experimental.pallas import tpu as pltpu
from jax.experimental.pallas import tpu_sc as plsc

N = 4_000_000
L = 16
VEC_LO = 7812
CHUNK_V = 868
CHUNK = CHUNK_V * L
NCHUNKS = VEC_LO // CHUNK_V
UNROLL = 14


def _body(z_hbm, tab_hbm, out_hbm, tab_v,
          z0, z1, g0, g1, in0, in1, out0, out1):
    cid = lax.axis_index("c")
    sid = lax.axis_index("s")
    wid = sid * 2 + cid
    base = (VEC_LO * wid + jnp.minimum(wid, 16)) * L

    z_bufs, g_bufs = (z0, z1), (g0, g1)
    in_sems, out_sems = (in0, in1), (out0, out1)

    pltpu.sync_copy(tab_hbm, tab_v)

    def in_copy(c, b):
        off = base + c * CHUNK
        return pltpu.make_async_copy(z_hbm.at[pl.ds(off, CHUNK)],
                                     z_bufs[b], in_sems[b])

    def out_copy(c, b):
        off = base + c * CHUNK
        return pltpu.make_async_copy(g_bufs[b],
                                     out_hbm.at[pl.ds(off, CHUNK)],
                                     out_sems[b])

    in_copy(0, 0).start()

    for c in range(NCHUNKS):
        b = c % 2
        if c + 1 < NCHUNKS:
            in_copy(c + 1, 1 - b).start()
        if c >= 2:
            out_copy(c - 2, b).wait()
        in_copy(c, b).wait()

        zb, gb = z_bufs[b], g_bufs[b]

        @plsc.parallel_loop(0, CHUNK, step=L, unroll=UNROLL)
        def _inner(s):
            zv = zb[pl.ds(s, L)]
            gb[pl.ds(s, L)] = plsc.load_gather(tab_v, [zv])

        out_copy(c, b).start()

    out_copy(NCHUNKS - 2, NCHUNKS % 2).wait()
    out_copy(NCHUNKS - 1, (NCHUNKS - 1) % 2).wait()

    @pl.when(wid < 16)
    def _tail():
        off = base + VEC_LO * L
        pltpu.sync_copy(z_hbm.at[pl.ds(off, L)], z0.at[pl.ds(0, L)])
        zv = z0[pl.ds(0, L)]
        g0[pl.ds(0, L)] = plsc.load_gather(tab_v, [zv])
        pltpu.sync_copy(g0.at[pl.ds(0, L)], out_hbm.at[pl.ds(off, L)])


def kernel(x, z, atomref_weight):
    zi = z.astype(jnp.int32)
    tab = atomref_weight.reshape(100)
    mesh = plsc.VectorSubcoreMesh(core_axis_name="c", subcore_axis_name="s")
    k = pl.kernel(
        _body,
        out_type=jax.ShapeDtypeStruct((N,), jnp.float32),
        mesh=mesh,
        compiler_params=pltpu.CompilerParams(
            needs_layout_passes=False, use_tc_tiling_on_sc=False),
        scratch_types=[
            pltpu.VMEM((100,), jnp.float32),
            pltpu.VMEM((CHUNK,), jnp.int32),
            pltpu.VMEM((CHUNK,), jnp.int32),
            pltpu.VMEM((CHUNK,), jnp.float32),
            pltpu.VMEM((CHUNK,), jnp.float32),
            pltpu.SemaphoreType.DMA,
            pltpu.SemaphoreType.DMA,
            pltpu.SemaphoreType.DMA,
            pltpu.SemaphoreType.DMA,
        ],
    )
    g = k(zi, tab)
    return x + g.reshape(N, 1)
